# Initial kernel scaffold; baseline (speedup 1.0000x reference)
import jax
import jax.numpy as jnp
from jax.experimental import pallas as pl


def kernel(x, pos, edge_index, cluster0, cluster1, cluster2, cluster3, cluster4, cluster5, W1, root1, b1, W2, root2, b2, W3, root3, b3, W4, root4, b4, W5, root5, b5, fc1_w, fc1_b, fc2_w, fc2_b):
    raise NotImplementedError("write your pallas kernel here")


# speedup vs baseline: 1.0619x; 1.0619x over previous
import functools
import math

import jax
import jax.numpy as jnp
from jax import lax
from jax.experimental import pallas as pl
from jax.experimental.pallas import tpu as pltpu

KS = 5
RADIX = (25, 5, 1)
OFFS = [(i, j, k) for i in (0, 1) for j in (0, 1) for k in (0, 1)]

LOG1P_SCALE = 30.0


def _edge_basis_body(d_ref, basis_ref, kidx_ref, *, inv_log1p_scale):
    d = d_ref[...]
    u = 0.5 + 0.5 * jnp.sign(d) * jnp.log1p(LOG1P_SCALE * jnp.abs(d)) * inv_log1p_scale
    u = jnp.clip(u, 0.0, 1.0)
    p = u * (KS - 1)
    bottom = jnp.clip(jnp.floor(p), 0.0, KS - 2)
    frac = p - bottom
    bot_i = bottom.astype(jnp.int32)
    b_rows = []
    k_rows = []
    for off in OFFS:
        b = jnp.ones_like(frac[0:1])
        k = jnp.zeros_like(bot_i[0:1])
        for dim in range(3):
            f = frac[dim:dim + 1]
            b = b * (f if off[dim] == 1 else (1.0 - f))
            k = k + (bot_i[dim:dim + 1] + off[dim]) * RADIX[dim]
        b_rows.append(b)
        k_rows.append(k)
    basis_ref[...] = jnp.concatenate(b_rows, axis=0)
    kidx_ref[...] = jnp.concatenate(k_rows, axis=0)


def _edge_basis(d3e):
    E = d3e.shape[1]
    blk = 1280
    assert E % blk == 0
    grid = (E // blk,)
    return pl.pallas_call(
        functools.partial(_edge_basis_body,
                          inv_log1p_scale=1.0 / math.log1p(LOG1P_SCALE)),
        grid=grid,
        in_specs=[pl.BlockSpec((3, blk), lambda i: (0, i))],
        out_specs=[pl.BlockSpec((8, blk), lambda i: (0, i)),
                   pl.BlockSpec((8, blk), lambda i: (0, i))],
        out_shape=[jax.ShapeDtypeStruct((8, E), jnp.float32),
                   jax.ShapeDtypeStruct((8, E), jnp.int32)],
    )(d3e)


def _xw_body(x_ref, w_ref, o_ref):
    o_ref[...] = jnp.dot(x_ref[...], w_ref[...],
                         preferred_element_type=jnp.float32)


def _xw_matmul(x, W):
    n, ci = x.shape
    K, _, co = W.shape
    Wf = W.transpose(1, 0, 2).reshape(ci, K * co)
    bn = K * co
    bm = 128 if bn >= 16000 else 256
    bm = min(bm, n)
    npad = (-n) % bm
    if npad:
        x = jnp.pad(x, ((0, npad), (0, 0)))
    M = x.shape[0]
    out = pl.pallas_call(
        _xw_body,
        grid=(M // bm,),
        in_specs=[pl.BlockSpec((bm, ci), lambda i: (i, 0)),
                  pl.BlockSpec((ci, bn), lambda i: (0, 0))],
        out_specs=pl.BlockSpec((bm, bn), lambda i: (i, 0)),
        out_shape=jax.ShapeDtypeStruct((M, K * co), jnp.float32),
    )(x, Wf)
    return out[:n]


def _combine_body(agg_ref, deg_ref, x_ref, root_ref, b_ref, o_ref):
    z = agg_ref[...] / jnp.maximum(deg_ref[...], 1.0)
    z = z + jnp.dot(x_ref[...], root_ref[...], preferred_element_type=jnp.float32)
    z = z + b_ref[...]
    o_ref[...] = jnp.where(z > 0, z, (jnp.exp(z) - 1.0))


def _combine(agg, deg, x, root, bias):
    n, co = agg.shape
    ci = x.shape[1]
    return pl.pallas_call(
        _combine_body,
        in_specs=[pl.BlockSpec((n, co), lambda: (0, 0)),
                  pl.BlockSpec((n, 1), lambda: (0, 0)),
                  pl.BlockSpec((n, ci), lambda: (0, 0)),
                  pl.BlockSpec((ci, co), lambda: (0, 0)),
                  pl.BlockSpec((1, co), lambda: (0, 0))],
        out_specs=pl.BlockSpec((n, co), lambda: (0, 0)),
        out_shape=jax.ShapeDtypeStruct((n, co), jnp.float32),
    )(agg, deg[:, None], x, root, bias[None, :])


def _head_body(z_ref, w1_ref, b1_ref, w2_ref, b2_ref, o_ref):
    z = z_ref[...].reshape(32, 1024)
    z = lax.dot_general(z, w1_ref[...], (((1,), (1,)), ((), ())),
                        preferred_element_type=jnp.float32) + b1_ref[...]
    z = jnp.where(z > 0, z, (jnp.exp(z) - 1.0))
    z = lax.dot_general(z, w2_ref[...], (((1,), (1,)), ((), ())),
                        preferred_element_type=jnp.float32) + b2_ref[...]
    m = jnp.max(z, axis=1, keepdims=True)
    s = z - m
    o_ref[...] = s - jnp.log(jnp.sum(jnp.exp(s), axis=1, keepdims=True))


def _head(hf, fc1_w, fc1_b, fc2_w, fc2_b):
    return pl.pallas_call(
        _head_body,
        in_specs=[pl.BlockSpec(hf.shape, lambda: (0, 0)),
                  pl.BlockSpec(fc1_w.shape, lambda: (0, 0)),
                  pl.BlockSpec((1, 512), lambda: (0, 0)),
                  pl.BlockSpec(fc2_w.shape, lambda: (0, 0)),
                  pl.BlockSpec((1, 10), lambda: (0, 0))],
        out_specs=pl.BlockSpec((32, 10), lambda: (0, 0)),
        out_shape=jax.ShapeDtypeStruct((32, 10), jnp.float32),
    )(hf, fc1_w, fc1_b[None, :], fc2_w, fc2_b[None, :])


def _graph_max_pool(x, pos, cluster, n_out):
    px = jax.ops.segment_max(x, cluster, num_segments=n_out)
    px = jnp.where(jnp.isfinite(px), px, 0.0)
    s = jax.ops.segment_sum(pos, cluster, num_segments=n_out)
    c = jax.ops.segment_sum(jnp.ones((pos.shape[0],), pos.dtype), cluster,
                            num_segments=n_out)
    return px, s / jnp.maximum(c, 1.0)[:, None]


def _spline_layer(h, ei, basis, kidx, W, root, bias, n):
    K, _, co = W.shape
    xw = _xw_matmul(h, W)
    xw = xw.reshape(n * K, co)
    src, dst = ei[0], ei[1]
    gi = src[None, :] * K + kidx
    msg = jnp.sum(xw[gi] * basis[..., None], axis=0)
    agg = jax.ops.segment_sum(msg, dst, num_segments=n)
    deg = jax.ops.segment_sum(jnp.ones(dst.shape, h.dtype), dst, num_segments=n)
    return _combine(agg, deg, h, root, bias)


def kernel(x, pos, edge_index, cluster0, cluster1, cluster2, cluster3,
           cluster4, cluster5, W1, root1, b1, W2, root2, b2, W3, root3, b3,
           W4, root4, b4, W5, root5, b5, fc1_w, fc1_b, fc2_w, fc2_b):
    sizes = [5000, 2500, 1250, 640, 320]
    convs = [(W1, root1, b1), (W2, root2, b2), (W3, root3, b3),
             (W4, root4, b4), (W5, root5, b5)]
    clusters = [cluster0, cluster1, cluster2, cluster3, cluster4]
    h, p, ei = x, pos, edge_index.astype(jnp.int32)
    for cl, n, (W, r, b) in zip(clusters, sizes, convs):
        cl = cl.astype(jnp.int32)
        h, p = _graph_max_pool(h, p, cl, n)
        ei = cl[ei]
        d = (p[ei[1]] - p[ei[0]]).T
        basis, kidx = _edge_basis(d)
        h = _spline_layer(h, ei, basis, kidx, W, r, b, n)
    hf, _ = _graph_max_pool(h, p, cluster5.astype(jnp.int32), 256)
    return _head(hf, fc1_w, fc1_b, fc2_w, fc2_b)


# speedup vs baseline: 3.5266x; 3.3209x over previous
import functools
import math

import jax
import jax.numpy as jnp
from jax import lax
from jax.experimental import pallas as pl
from jax.experimental.pallas import tpu as pltpu
from jax.experimental.pallas import tpu_sc as plsc

KS = 5
RADIX = (25, 5, 1)
OFFS = [(i, j, k) for i in (0, 1) for j in (0, 1) for k in (0, 1)]

LOG1P_SCALE = 30.0


def _edge_basis_body(d_ref, src_ref, basis_ref, *gidx_refs, K, S,
                     inv_log1p_scale):
    d = d_ref[...]
    u = 0.5 + 0.5 * jnp.sign(d) * jnp.log1p(LOG1P_SCALE * jnp.abs(d)) * inv_log1p_scale
    u = jnp.clip(u, 0.0, 1.0)
    p = u * (KS - 1)
    bottom = jnp.clip(jnp.floor(p), 0.0, KS - 2)
    frac = p - bottom
    bot_i = bottom.astype(jnp.int32)
    b_rows = []
    k_rows = []
    for off in OFFS:
        b = jnp.ones_like(frac[0:1])
        k = jnp.zeros_like(bot_i[0:1])
        for dim in range(3):
            f = frac[dim:dim + 1]
            b = b * (f if off[dim] == 1 else (1.0 - f))
            k = k + (bot_i[dim:dim + 1] + off[dim]) * RADIX[dim]
        b_rows.append(b)
        k_rows.append(k)
    basis_ref[...] = jnp.concatenate(b_rows, axis=0)
    gi = jnp.concatenate(k_rows, axis=0) + src_ref[...] * K
    for h in range(S):
        gidx_refs[h][...] = gi * S + h


def _edge_basis(d3e, src, K, S):
    E = d3e.shape[1]
    blk = 1280
    grid = (E // blk,)
    outs = pl.pallas_call(
        functools.partial(_edge_basis_body, K=K, S=S,
                          inv_log1p_scale=1.0 / math.log1p(LOG1P_SCALE)),
        grid=grid,
        in_specs=[pl.BlockSpec((3, blk), lambda i: (0, i)),
                  pl.BlockSpec((1, blk), lambda i: (0, i))],
        out_specs=[pl.BlockSpec((8, blk), lambda i: (0, i))] * (1 + S),
        out_shape=[jax.ShapeDtypeStruct((8, E), jnp.float32)]
        + [jax.ShapeDtypeStruct((8, E), jnp.int32)] * S,
    )(d3e, src)
    return outs[0], outs[1:]


def _make_sc_edge_agg(n_pad, co, E):
    W = co + 16
    G = co // 16
    NW = 32
    EW = E // NW
    C = 40
    nchunk = EW // C
    stripe = n_pad // 16
    mesh = plsc.VectorSubcoreMesh(core_axis_name="c", subcore_axis_name="s",
                                  num_cores=2, num_subcores=16)

    @functools.partial(
        pl.kernel,
        out_type=jax.ShapeDtypeStruct((2, n_pad, W), jnp.float32),
        mesh=mesh,
        scratch_types=[
            pltpu.VMEM((8, C), jnp.int32),
            pltpu.VMEM((8, C + 16), jnp.float32),
            pltpu.VMEM((C,), jnp.int32),
            pltpu.VMEM((8, C, co), jnp.float32),
            pltpu.VMEM((C, W), jnp.float32),
            pltpu.VMEM((stripe, W), jnp.float32),
            pltpu.VMEM_SHARED((n_pad, W), jnp.float32),
            pltpu.SemaphoreType.DMA,
        ],
        compiler_params=pltpu.CompilerParams(use_tc_tiling_on_sc=False),
    )
    def k(xw_hbm, gidx_hbm, basis_hbm, dst_hbm, out_hbm,
          idx_v, bas_v, dst_v, rows_v, msg_v, zero_v, agg_sh, sem):
        cid = lax.axis_index("c")
        sid = lax.axis_index("s")
        wid = sid * 2 + cid

        zvec = jnp.zeros((16,), jnp.float32)
        one_vec = jnp.where(lax.iota(jnp.int32, 16) == 0, 1.0, 0.0)

        def zrow(r, carry):
            for wg in range(W // 16):
                zero_v[r, pl.ds(wg * 16, 16)] = zvec
            return carry
        lax.fori_loop(0, stripe, zrow, 0)
        pltpu.sync_copy(zero_v, agg_sh.at[pl.ds(sid * stripe, stripe)])
        plsc.subcore_barrier()

        base0 = wid * EW

        def chunk(g, carry):
            base = base0 + g * C
            for j in range(8):
                pltpu.sync_copy(gidx_hbm.at[pl.ds(j * E + base, C)],
                                idx_v.at[j])
                pltpu.sync_copy(basis_hbm.at[pl.ds(j * E + base, C)],
                                bas_v.at[j, pl.ds(0, C)])
            pltpu.sync_copy(dst_hbm.at[pl.ds(base, C)], dst_v)
            cps = [pltpu.async_copy(xw_hbm.at[idx_v.at[j]], rows_v.at[j], sem)
                   for j in range(8)]
            for cp in cps:
                cp.wait()

            def edge(e, c2):
                bv = [bas_v[j, pl.ds(e, 16)][0] for j in range(8)]
                for gg in range(G):
                    acc = bv[0] * rows_v[0, e, pl.ds(gg * 16, 16)]
                    for j in range(1, 8):
                        acc = acc + bv[j] * rows_v[j, e, pl.ds(gg * 16, 16)]
                    msg_v[e, pl.ds(gg * 16, 16)] = acc
                msg_v[e, pl.ds(co, 16)] = one_vec
                return c2
            lax.fori_loop(0, C, edge, 0)
            pltpu.sync_copy(msg_v, agg_sh.at[dst_v], add=True)
            return carry
        lax.fori_loop(0, nchunk, chunk, 0)

        plsc.subcore_barrier()
        @pl.when(sid == 0)
        def _dump():
            pltpu.sync_copy(agg_sh, out_hbm.at[cid])

    return k


def _xw_body(x_ref, w_ref, o_ref):
    o_ref[...] = jnp.dot(x_ref[...], w_ref[...],
                         preferred_element_type=jnp.float32)


def _xw_matmul(x, W):
    n, ci = x.shape
    K, _, co = W.shape
    Wf = W.transpose(1, 0, 2).reshape(ci, K * co)
    bn = K * co
    bm = 128 if bn >= 16000 else 256
    bm = min(bm, n)
    npad = (-n) % bm
    if npad:
        x = jnp.pad(x, ((0, npad), (0, 0)))
    M = x.shape[0]
    out = pl.pallas_call(
        _xw_body,
        grid=(M // bm,),
        in_specs=[pl.BlockSpec((bm, ci), lambda i: (i, 0)),
                  pl.BlockSpec((ci, bn), lambda i: (0, 0))],
        out_specs=pl.BlockSpec((bm, bn), lambda i: (i, 0)),
        out_shape=jax.ShapeDtypeStruct((M, K * co), jnp.float32),
    )(x, Wf)
    return out[:n]


def _combine_body(x_ref, root_ref, b_ref, *refs, n, co_eff, S):
    t_refs, o_ref = refs[:S], refs[S]
    parts = []
    for h in range(S):
        t = t_refs[h][...]
        parts.append(t[0, :n, :co_eff] + t[1, :n, :co_eff])
    agg = jnp.concatenate(parts, axis=1) if S > 1 else parts[0]
    t0 = t_refs[0][...]
    deg = (t0[0, :n, co_eff] + t0[1, :n, co_eff])[:, None]
    z = agg / jnp.maximum(deg, 1.0)
    z = z + jnp.dot(x_ref[...], root_ref[...], preferred_element_type=jnp.float32)
    z = z + b_ref[...]
    o_ref[...] = jnp.where(z > 0, z, (jnp.exp(z) - 1.0))


def _combine(tables, x, root, bias, n, co_eff, S):
    n_pad, Wt = tables[0].shape[1], tables[0].shape[2]
    ci, co = root.shape
    return pl.pallas_call(
        functools.partial(_combine_body, n=n, co_eff=co_eff, S=S),
        in_specs=[pl.BlockSpec((n, ci), lambda: (0, 0)),
                  pl.BlockSpec((ci, co), lambda: (0, 0)),
                  pl.BlockSpec((1, co), lambda: (0, 0))]
        + [pl.BlockSpec((2, n_pad, Wt), lambda: (0, 0, 0))] * S,
        out_specs=pl.BlockSpec((n, co), lambda: (0, 0)),
        out_shape=jax.ShapeDtypeStruct((n, co), jnp.float32),
    )(x, root, bias[None, :], *tables)


def _head_body(z_ref, w1_ref, b1_ref, w2_ref, b2_ref, o_ref):
    z = z_ref[...].reshape(32, 1024)
    z = lax.dot_general(z, w1_ref[...], (((1,), (1,)), ((), ())),
                        preferred_element_type=jnp.float32) + b1_ref[...]
    z = jnp.where(z > 0, z, (jnp.exp(z) - 1.0))
    z = lax.dot_general(z, w2_ref[...], (((1,), (1,)), ((), ())),
                        preferred_element_type=jnp.float32) + b2_ref[...]
    m = jnp.max(z, axis=1, keepdims=True)
    s = z - m
    o_ref[...] = s - jnp.log(jnp.sum(jnp.exp(s), axis=1, keepdims=True))


def _head(hf, fc1_w, fc1_b, fc2_w, fc2_b):
    return pl.pallas_call(
        _head_body,
        in_specs=[pl.BlockSpec(hf.shape, lambda: (0, 0)),
                  pl.BlockSpec(fc1_w.shape, lambda: (0, 0)),
                  pl.BlockSpec((1, 512), lambda: (0, 0)),
                  pl.BlockSpec(fc2_w.shape, lambda: (0, 0)),
                  pl.BlockSpec((1, 10), lambda: (0, 0))],
        out_specs=pl.BlockSpec((32, 10), lambda: (0, 0)),
        out_shape=jax.ShapeDtypeStruct((32, 10), jnp.float32),
    )(hf, fc1_w, fc1_b[None, :], fc2_w, fc2_b[None, :])


def _graph_max_pool(x, pos, cluster, n_out):
    px = jax.ops.segment_max(x, cluster, num_segments=n_out)
    px = jnp.where(jnp.isfinite(px), px, 0.0)
    s = jax.ops.segment_sum(pos, cluster, num_segments=n_out)
    c = jax.ops.segment_sum(jnp.ones((pos.shape[0],), pos.dtype), cluster,
                            num_segments=n_out)
    return px, s / jnp.maximum(c, 1.0)[:, None]


def _n_pad(n):
    stripe = -(-n // 128) * 8
    return stripe * 16


def _spline_layer(h, ei, p, W, root, bias, n):
    K, ci, co = W.shape
    E = ei.shape[1]
    S = 2 if co > 64 else 1
    co_eff = co // S
    src, dst = ei[0], ei[1]
    d = (p[ei[1]] - p[ei[0]]).T
    basis, gidxs = _edge_basis(d, src[None, :], K, S)
    xw = _xw_matmul(h, W)
    xw2 = xw.reshape(n * K * S, co_eff)
    npad = _n_pad(n)
    sc_agg = _make_sc_edge_agg(npad, co_eff, E)
    basis_f = basis.reshape(8 * E)
    tables = [sc_agg(xw2, gidxs[hh].reshape(8 * E), basis_f, dst)
              for hh in range(S)]
    return _combine(tables, h, root, bias, n, co_eff, S)


def kernel(x, pos, edge_index, cluster0, cluster1, cluster2, cluster3,
           cluster4, cluster5, W1, root1, b1, W2, root2, b2, W3, root3, b3,
           W4, root4, b4, W5, root5, b5, fc1_w, fc1_b, fc2_w, fc2_b):
    sizes = [5000, 2500, 1250, 640, 320]
    convs = [(W1, root1, b1), (W2, root2, b2), (W3, root3, b3),
             (W4, root4, b4), (W5, root5, b5)]
    clusters = [cluster0, cluster1, cluster2, cluster3, cluster4]
    h, p, ei = x, pos, edge_index.astype(jnp.int32)
    for cl, n, (W, r, b) in zip(clusters, sizes, convs):
        cl = cl.astype(jnp.int32)
        h, p = _graph_max_pool(h, p, cl, n)
        ei = cl[ei]
        h = _spline_layer(h, ei, p, W, r, b, n)
    hf, _ = _graph_max_pool(h, p, cluster5.astype(jnp.int32), 256)
    return _head(hf, fc1_w, fc1_b, fc2_w, fc2_b)


# speedup vs baseline: 5.7534x; 1.6314x over previous
import functools
import math

import jax
import jax.numpy as jnp
from jax import lax
from jax.experimental import pallas as pl
from jax.experimental.pallas import tpu as pltpu
from jax.experimental.pallas import tpu_sc as plsc

KS = 5
RADIX = (25, 5, 1)
OFFS = [(i, j, k) for i in (0, 1) for j in (0, 1) for k in (0, 1)]

LOG1P_SCALE = 30.0


def _edge_basis_body(d_ref, src_ref, basis_ref, *gidx_refs, K, S,
                     inv_log1p_scale):
    d = d_ref[...]
    u = 0.5 + 0.5 * jnp.sign(d) * jnp.log1p(LOG1P_SCALE * jnp.abs(d)) * inv_log1p_scale
    u = jnp.clip(u, 0.0, 1.0)
    p = u * (KS - 1)
    bottom = jnp.clip(jnp.floor(p), 0.0, KS - 2)
    frac = p - bottom
    bot_i = bottom.astype(jnp.int32)
    b_rows = []
    k_rows = []
    for off in OFFS:
        b = jnp.ones_like(frac[0:1])
        k = jnp.zeros_like(bot_i[0:1])
        for dim in range(3):
            f = frac[dim:dim + 1]
            b = b * (f if off[dim] == 1 else (1.0 - f))
            k = k + (bot_i[dim:dim + 1] + off[dim]) * RADIX[dim]
        b_rows.append(b)
        k_rows.append(k)
    basis_ref[...] = jnp.concatenate(b_rows, axis=0)
    gi = jnp.concatenate(k_rows, axis=0) + src_ref[...] * K
    for h in range(S):
        gidx_refs[h][...] = gi * S + h


def _edge_basis(d3e, src, K, S):
    E = d3e.shape[1]
    blk = 1280
    grid = (E // blk,)
    outs = pl.pallas_call(
        functools.partial(_edge_basis_body, K=K, S=S,
                          inv_log1p_scale=1.0 / math.log1p(LOG1P_SCALE)),
        grid=grid,
        in_specs=[pl.BlockSpec((3, blk), lambda i: (0, i)),
                  pl.BlockSpec((1, blk), lambda i: (0, i))],
        out_specs=[pl.BlockSpec((8, blk), lambda i: (0, i))] * (1 + S),
        out_shape=[jax.ShapeDtypeStruct((8, E), jnp.float32)]
        + [jax.ShapeDtypeStruct((8, E), jnp.int32)] * S,
    )(d3e, src)
    return outs[0], outs[1:]


def _make_sc_geometry(prev_n, n, E):
    C = 640
    NCHUNK = E // C
    mesh = plsc.VectorSubcoreMesh(core_axis_name="c", subcore_axis_name="s",
                                  num_cores=2, num_subcores=16)

    @functools.partial(
        pl.kernel,
        out_type=[jax.ShapeDtypeStruct((E,), jnp.int32),
                  jax.ShapeDtypeStruct((E,), jnp.int32),
                  jax.ShapeDtypeStruct((3 * E,), jnp.float32)],
        mesh=mesh,
        scratch_types=[
            pltpu.VMEM((prev_n,), jnp.int32),
            pltpu.VMEM((3, n), jnp.float32),
            pltpu.VMEM((C,), jnp.int32),
            pltpu.VMEM((C,), jnp.int32),
            pltpu.VMEM((C,), jnp.int32),
            pltpu.VMEM((C,), jnp.int32),
            pltpu.VMEM((3, C), jnp.float32),
        ],
        compiler_params=pltpu.CompilerParams(use_tc_tiling_on_sc=False,
                                             needs_layout_passes=False),
    )
    def k(cl_hbm, post_hbm, srcp_hbm, dstp_hbm, nsrc_hbm, ndst_hbm, d_hbm,
          cl_v, pos_v, sv, dv, nsv, ndv, dbuf):
        cid = lax.axis_index("c")
        sid = lax.axis_index("s")
        wid = sid * 2 + cid
        pltpu.sync_copy(cl_hbm, cl_v)
        pltpu.sync_copy(post_hbm, pos_v)
        nfull, rem = NCHUNK // 32, NCHUNK % 32
        ngroups = nfull + jnp.where(wid < rem, 1, 0)

        def chunk(g, carry):
            base = (wid + g * 32) * C
            pltpu.sync_copy(srcp_hbm.at[pl.ds(base, C)], sv)
            pltpu.sync_copy(dstp_hbm.at[pl.ds(base, C)], dv)
            for t in range(C // 16):
                sl = pl.ds(t * 16, 16)
                ns = plsc.load_gather(cl_v, [sv[sl]])
                nd = plsc.load_gather(cl_v, [dv[sl]])
                nsv[sl] = ns
                ndv[sl] = nd
                for dim in range(3):
                    dimv = jnp.full((16,), dim, jnp.int32)
                    ps = plsc.load_gather(pos_v, [dimv, ns])
                    pd = plsc.load_gather(pos_v, [dimv, nd])
                    dbuf[dim, sl] = pd - ps
            pltpu.sync_copy(nsv, nsrc_hbm.at[pl.ds(base, C)])
            pltpu.sync_copy(ndv, ndst_hbm.at[pl.ds(base, C)])
            for dim in range(3):
                pltpu.sync_copy(dbuf.at[dim], d_hbm.at[pl.ds(dim * E + base, C)])
            return carry
        lax.fori_loop(0, ngroups, chunk, 0)

    return k


def _make_sc_edge_agg(n_pad, co, E):
    W = co + 16
    G = co // 16
    NW = 32
    EW = E // NW
    C = 40
    nchunk = EW // C
    stripe = n_pad // 16
    mesh = plsc.VectorSubcoreMesh(core_axis_name="c", subcore_axis_name="s",
                                  num_cores=2, num_subcores=16)

    @functools.partial(
        pl.kernel,
        out_type=jax.ShapeDtypeStruct((2, n_pad, W), jnp.float32),
        mesh=mesh,
        scratch_types=[
            pltpu.VMEM((8, C), jnp.int32),
            pltpu.VMEM((8, C + 16), jnp.float32),
            pltpu.VMEM((C,), jnp.int32),
            pltpu.VMEM((8, C, co), jnp.float32),
            pltpu.VMEM((C, W), jnp.float32),
            pltpu.VMEM((stripe, W), jnp.float32),
            pltpu.VMEM_SHARED((n_pad, W), jnp.float32),
            pltpu.SemaphoreType.DMA,
        ],
        compiler_params=pltpu.CompilerParams(use_tc_tiling_on_sc=False),
    )
    def k(xw_hbm, gidx_hbm, basis_hbm, dst_hbm, out_hbm,
          idx_v, bas_v, dst_v, rows_v, msg_v, zero_v, agg_sh, sem):
        cid = lax.axis_index("c")
        sid = lax.axis_index("s")
        wid = sid * 2 + cid

        zvec = jnp.zeros((16,), jnp.float32)
        one_vec = jnp.where(lax.iota(jnp.int32, 16) == 0, 1.0, 0.0)

        def zrow(r, carry):
            for wg in range(W // 16):
                zero_v[r, pl.ds(wg * 16, 16)] = zvec
            return carry
        lax.fori_loop(0, stripe, zrow, 0)
        pltpu.sync_copy(zero_v, agg_sh.at[pl.ds(sid * stripe, stripe)])
        plsc.subcore_barrier()

        base0 = wid * EW

        def chunk(g, carry):
            base = base0 + g * C
            for j in range(8):
                pltpu.sync_copy(gidx_hbm.at[pl.ds(j * E + base, C)],
                                idx_v.at[j])
                pltpu.sync_copy(basis_hbm.at[pl.ds(j * E + base, C)],
                                bas_v.at[j, pl.ds(0, C)])
            pltpu.sync_copy(dst_hbm.at[pl.ds(base, C)], dst_v)
            cps = [pltpu.async_copy(xw_hbm.at[idx_v.at[j]], rows_v.at[j], sem)
                   for j in range(8)]
            for cp in cps:
                cp.wait()

            def edge(e, c2):
                bv = [bas_v[j, pl.ds(e, 16)][0] for j in range(8)]
                for gg in range(G):
                    acc = bv[0] * rows_v[0, e, pl.ds(gg * 16, 16)]
                    for j in range(1, 8):
                        acc = acc + bv[j] * rows_v[j, e, pl.ds(gg * 16, 16)]
                    msg_v[e, pl.ds(gg * 16, 16)] = acc
                msg_v[e, pl.ds(co, 16)] = one_vec
                return c2
            lax.fori_loop(0, C, edge, 0)
            pltpu.sync_copy(msg_v, agg_sh.at[dst_v], add=True)
            return carry
        lax.fori_loop(0, nchunk, chunk, 0)

        plsc.subcore_barrier()
        @pl.when(sid == 0)
        def _dump():
            pltpu.sync_copy(agg_sh, out_hbm.at[cid])

    return k


def _xw_body(x_ref, w_ref, o_ref):
    o_ref[...] = jnp.dot(x_ref[...], w_ref[...],
                         preferred_element_type=jnp.float32)


def _xw_matmul(x, W):
    n, ci = x.shape
    K, _, co = W.shape
    Wf = W.transpose(1, 0, 2).reshape(ci, K * co)
    bn = K * co
    bm = 128 if bn >= 16000 else 256
    bm = min(bm, n)
    npad = (-n) % bm
    if npad:
        x = jnp.pad(x, ((0, npad), (0, 0)))
    M = x.shape[0]
    out = pl.pallas_call(
        _xw_body,
        grid=(M // bm,),
        in_specs=[pl.BlockSpec((bm, ci), lambda i: (i, 0)),
                  pl.BlockSpec((ci, bn), lambda i: (0, 0))],
        out_specs=pl.BlockSpec((bm, bn), lambda i: (i, 0)),
        out_shape=jax.ShapeDtypeStruct((M, K * co), jnp.float32),
    )(x, Wf)
    return out[:n]


def _combine_body(x_ref, root_ref, b_ref, *refs, n, co_eff, S):
    t_refs, o_ref = refs[:S], refs[S]
    parts = []
    for h in range(S):
        t = t_refs[h][...]
        parts.append(t[0, :n, :co_eff] + t[1, :n, :co_eff])
    agg = jnp.concatenate(parts, axis=1) if S > 1 else parts[0]
    t0 = t_refs[0][...]
    deg = (t0[0, :n, co_eff] + t0[1, :n, co_eff])[:, None]
    z = agg / jnp.maximum(deg, 1.0)
    z = z + jnp.dot(x_ref[...], root_ref[...], preferred_element_type=jnp.float32)
    z = z + b_ref[...]
    o_ref[...] = jnp.where(z > 0, z, (jnp.exp(z) - 1.0))


def _combine(tables, x, root, bias, n, co_eff, S):
    n_pad, Wt = tables[0].shape[1], tables[0].shape[2]
    ci, co = root.shape
    return pl.pallas_call(
        functools.partial(_combine_body, n=n, co_eff=co_eff, S=S),
        in_specs=[pl.BlockSpec((n, ci), lambda: (0, 0)),
                  pl.BlockSpec((ci, co), lambda: (0, 0)),
                  pl.BlockSpec((1, co), lambda: (0, 0))]
        + [pl.BlockSpec((2, n_pad, Wt), lambda: (0, 0, 0))] * S,
        out_specs=pl.BlockSpec((n, co), lambda: (0, 0)),
        out_shape=jax.ShapeDtypeStruct((n, co), jnp.float32),
    )(x, root, bias[None, :], *tables)


def _head_body(z_ref, w1_ref, b1_ref, w2_ref, b2_ref, o_ref):
    z = z_ref[...].reshape(32, 1024)
    z = lax.dot_general(z, w1_ref[...], (((1,), (1,)), ((), ())),
                        preferred_element_type=jnp.float32) + b1_ref[...]
    z = jnp.where(z > 0, z, (jnp.exp(z) - 1.0))
    z = lax.dot_general(z, w2_ref[...], (((1,), (1,)), ((), ())),
                        preferred_element_type=jnp.float32) + b2_ref[...]
    m = jnp.max(z, axis=1, keepdims=True)
    s = z - m
    o_ref[...] = s - jnp.log(jnp.sum(jnp.exp(s), axis=1, keepdims=True))


def _head(hf, fc1_w, fc1_b, fc2_w, fc2_b):
    return pl.pallas_call(
        _head_body,
        in_specs=[pl.BlockSpec(hf.shape, lambda: (0, 0)),
                  pl.BlockSpec(fc1_w.shape, lambda: (0, 0)),
                  pl.BlockSpec((1, 512), lambda: (0, 0)),
                  pl.BlockSpec(fc2_w.shape, lambda: (0, 0)),
                  pl.BlockSpec((1, 10), lambda: (0, 0))],
        out_specs=pl.BlockSpec((32, 10), lambda: (0, 0)),
        out_shape=jax.ShapeDtypeStruct((32, 10), jnp.float32),
    )(hf, fc1_w, fc1_b[None, :], fc2_w, fc2_b[None, :])


def _graph_max_pool(x, pos, cluster, n_out):
    px = jax.ops.segment_max(x, cluster, num_segments=n_out)
    px = jnp.where(jnp.isfinite(px), px, 0.0)
    s = jax.ops.segment_sum(pos, cluster, num_segments=n_out)
    c = jax.ops.segment_sum(jnp.ones((pos.shape[0],), pos.dtype), cluster,
                            num_segments=n_out)
    return px, s / jnp.maximum(c, 1.0)[:, None]


def _n_pad(n):
    stripe = -(-n // 128) * 8
    return stripe * 16


def _spline_layer(h, src, dst, d3e, W, root, bias, n):
    K, ci, co = W.shape
    E = src.shape[0]
    S = 2 if co > 64 else 1
    co_eff = co // S
    basis, gidxs = _edge_basis(d3e, src[None, :], K, S)
    xw = _xw_matmul(h, W)
    xw2 = xw.reshape(n * K * S, co_eff)
    npad = _n_pad(n)
    sc_agg = _make_sc_edge_agg(npad, co_eff, E)
    basis_f = basis.reshape(8 * E)
    tables = [sc_agg(xw2, gidxs[hh].reshape(8 * E), basis_f, dst)
              for hh in range(S)]
    return _combine(tables, h, root, bias, n, co_eff, S)


def kernel(x, pos, edge_index, cluster0, cluster1, cluster2, cluster3,
           cluster4, cluster5, W1, root1, b1, W2, root2, b2, W3, root3, b3,
           W4, root4, b4, W5, root5, b5, fc1_w, fc1_b, fc2_w, fc2_b):
    sizes = [5000, 2500, 1250, 640, 320]
    convs = [(W1, root1, b1), (W2, root2, b2), (W3, root3, b3),
             (W4, root4, b4), (W5, root5, b5)]
    clusters = [cluster0, cluster1, cluster2, cluster3, cluster4]
    h, p = x, pos
    ei32 = edge_index.astype(jnp.int32)
    src, dst = ei32[0], ei32[1]
    E = src.shape[0]
    prev_n = x.shape[0]
    for cl, n, (W, r, b) in zip(clusters, sizes, convs):
        cl = cl.astype(jnp.int32)
        h, p = _graph_max_pool(h, p, cl, n)
        geom = _make_sc_geometry(prev_n, n, E)
        src, dst, dflat = geom(cl, p.T, src, dst)
        h = _spline_layer(h, src, dst, dflat.reshape(3, E), W, r, b, n)
        prev_n = n
    hf, _ = _graph_max_pool(h, p, cluster5.astype(jnp.int32), 256)
    return _head(hf, fc1_w, fc1_b, fc2_w, fc2_b)


# speedup vs baseline: 13.5978x; 2.3634x over previous
import functools
import math

import jax
import jax.numpy as jnp
from jax import lax
from jax.experimental import pallas as pl
from jax.experimental.pallas import tpu as pltpu
from jax.experimental.pallas import tpu_sc as plsc

KS = 5
RADIX = (25, 5, 1)
OFFS = [(i, j, k) for i in (0, 1) for j in (0, 1) for k in (0, 1)]

LOG1P_SCALE = 30.0


def _edge_meta_body(d_ref, src_ref, dst_ref, *meta_refs, K, S,
                    inv_log1p_scale):
    d = d_ref[...]
    u = 0.5 + 0.5 * jnp.sign(d) * jnp.log1p(LOG1P_SCALE * jnp.abs(d)) * inv_log1p_scale
    u = jnp.clip(u, 0.0, 1.0)
    p = u * (KS - 1)
    bottom = jnp.clip(jnp.floor(p), 0.0, KS - 2)
    frac = p - bottom
    bot_i = bottom.astype(jnp.int32)
    b_rows = []
    k_rows = []
    for off in OFFS:
        b = jnp.ones_like(frac[0:1])
        k = jnp.zeros_like(bot_i[0:1])
        for dim in range(3):
            f = frac[dim:dim + 1]
            b = b * (f if off[dim] == 1 else (1.0 - f))
            k = k + (bot_i[dim:dim + 1] + off[dim]) * RADIX[dim]
        b_rows.append(b)
        k_rows.append(k)
    bas_i = lax.bitcast_convert_type(jnp.concatenate(b_rows, axis=0),
                                     jnp.int32)
    gi = jnp.concatenate(k_rows, axis=0) + src_ref[...] * K
    for h in range(S):
        meta_refs[h][...] = jnp.concatenate(
            [gi * S + h, bas_i, dst_ref[...]], axis=0)


def _edge_meta(d3e, src, dst, K, S):
    E = d3e.shape[1]
    blk = 1280
    grid = (E // blk,)
    outs = pl.pallas_call(
        functools.partial(_edge_meta_body, K=K, S=S,
                          inv_log1p_scale=1.0 / math.log1p(LOG1P_SCALE)),
        grid=grid,
        in_specs=[pl.BlockSpec((3, blk), lambda i: (0, i)),
                  pl.BlockSpec((1, blk), lambda i: (0, i)),
                  pl.BlockSpec((1, blk), lambda i: (0, i))],
        out_specs=[pl.BlockSpec((17, blk), lambda i: (0, i))] * S,
        out_shape=[jax.ShapeDtypeStruct((17, E), jnp.int32)] * S,
    )(d3e, src, dst)
    return outs


def _make_sc_geometry(prev_n, n, E):
    C = 640
    NCHUNK = E // C
    mesh = plsc.VectorSubcoreMesh(core_axis_name="c", subcore_axis_name="s",
                                  num_cores=2, num_subcores=16)

    @functools.partial(
        pl.kernel,
        out_type=[jax.ShapeDtypeStruct((E,), jnp.int32),
                  jax.ShapeDtypeStruct((E,), jnp.int32),
                  jax.ShapeDtypeStruct((3 * E,), jnp.float32)],
        mesh=mesh,
        scratch_types=[
            pltpu.VMEM((prev_n,), jnp.int32),
            pltpu.VMEM((3, n), jnp.float32),
            pltpu.VMEM((C,), jnp.int32),
            pltpu.VMEM((C,), jnp.int32),
            pltpu.VMEM((C,), jnp.int32),
            pltpu.VMEM((C,), jnp.int32),
            pltpu.VMEM((3, C), jnp.float32),
        ],
        compiler_params=pltpu.CompilerParams(use_tc_tiling_on_sc=False,
                                             needs_layout_passes=False),
    )
    def k(cl_hbm, post_hbm, srcp_hbm, dstp_hbm, nsrc_hbm, ndst_hbm, d_hbm,
          cl_v, pos_v, sv, dv, nsv, ndv, dbuf):
        cid = lax.axis_index("c")
        sid = lax.axis_index("s")
        wid = sid * 2 + cid
        pltpu.sync_copy(cl_hbm, cl_v)
        pltpu.sync_copy(post_hbm, pos_v)
        nfull, rem = NCHUNK // 32, NCHUNK % 32
        ngroups = nfull + jnp.where(wid < rem, 1, 0)

        def chunk(g, carry):
            base = (wid + g * 32) * C
            pltpu.sync_copy(srcp_hbm.at[pl.ds(base, C)], sv)
            pltpu.sync_copy(dstp_hbm.at[pl.ds(base, C)], dv)
            for t in range(C // 16):
                sl = pl.ds(t * 16, 16)
                ns = plsc.load_gather(cl_v, [sv[sl]])
                nd = plsc.load_gather(cl_v, [dv[sl]])
                nsv[sl] = ns
                ndv[sl] = nd
                for dim in range(3):
                    dimv = jnp.full((16,), dim, jnp.int32)
                    ps = plsc.load_gather(pos_v, [dimv, ns])
                    pd = plsc.load_gather(pos_v, [dimv, nd])
                    dbuf[dim, sl] = pd - ps
            pltpu.sync_copy(nsv, nsrc_hbm.at[pl.ds(base, C)])
            pltpu.sync_copy(ndv, ndst_hbm.at[pl.ds(base, C)])
            for dim in range(3):
                pltpu.sync_copy(dbuf.at[dim], d_hbm.at[pl.ds(dim * E + base, C)])
            return carry
        lax.fori_loop(0, ngroups, chunk, 0)

    return k


def _make_sc_edge_agg(n_pad, co, E):
    W = co + 16
    G = co // 16
    C = 128
    NCHUNK = E // C
    stripe = n_pad // 16
    mesh = plsc.VectorSubcoreMesh(core_axis_name="c", subcore_axis_name="s",
                                  num_cores=2, num_subcores=16)

    @functools.partial(
        pl.kernel,
        out_type=jax.ShapeDtypeStruct((2, n_pad, W), jnp.float32),
        mesh=mesh,
        scratch_types=[
            pltpu.VMEM((17, C), jnp.int32),
            pltpu.VMEM((8, C, co), jnp.float32),
            pltpu.VMEM((C, W), jnp.float32),
            pltpu.VMEM((stripe, W), jnp.float32),
            pltpu.VMEM_SHARED((n_pad, W), jnp.float32),
            pltpu.SemaphoreType.DMA,
        ],
        compiler_params=pltpu.CompilerParams(use_tc_tiling_on_sc=False,
                                             needs_layout_passes=False),
    )
    def k(xw_hbm, meta_hbm, out_hbm, meta_v, rows_v, msg_v, zero_v, agg_sh,
          sem):
        cid = lax.axis_index("c")
        sid = lax.axis_index("s")
        wid = sid * 2 + cid

        zvec = jnp.zeros((16,), jnp.float32)
        one_vec = jnp.where(lax.iota(jnp.int32, 16) == 0, 1.0, 0.0)

        def zrow(r, carry):
            for wg in range(W // 16):
                zero_v[r, pl.ds(wg * 16, 16)] = zvec
            return carry
        lax.fori_loop(0, stripe, zrow, 0)
        pltpu.sync_copy(zero_v, agg_sh.at[pl.ds(sid * stripe, stripe)])
        plsc.subcore_barrier()

        nfull, rem = NCHUNK // 32, NCHUNK % 32
        ngroups = nfull + jnp.where(wid < rem, 1, 0)

        def chunk(g, carry):
            base = (wid + g * 32) * C
            pltpu.sync_copy(meta_hbm.at[:, pl.ds(base, C)], meta_v)
            cps = [pltpu.async_copy(xw_hbm.at[meta_v.at[j]], rows_v.at[j], sem)
                   for j in range(8)]
            for cp in cps:
                cp.wait()

            def group(t, c2):
                e0 = t * 16
                bvecs = [plsc.bitcast(meta_v[8 + j, pl.ds(e0, 16)],
                                      jnp.float32) for j in range(8)]
                for l in range(16):
                    e = e0 + l
                    for gg in range(G):
                        acc = bvecs[0][l] * rows_v[0, e, pl.ds(gg * 16, 16)]
                        for j in range(1, 8):
                            acc = acc + bvecs[j][l] * rows_v[j, e, pl.ds(gg * 16, 16)]
                        msg_v[e, pl.ds(gg * 16, 16)] = acc
                    msg_v[e, pl.ds(co, 16)] = one_vec
                return c2
            lax.fori_loop(0, C // 16, group, 0)
            pltpu.sync_copy(msg_v, agg_sh.at[meta_v.at[16]], add=True)
            return carry
        lax.fori_loop(0, ngroups, chunk, 0)

        plsc.subcore_barrier()
        @pl.when(sid == 0)
        def _dump():
            pltpu.sync_copy(agg_sh, out_hbm.at[cid])

    return k


def _xw_body(x_ref, w_ref, o_ref):
    o_ref[...] = jnp.dot(x_ref[...], w_ref[...],
                         preferred_element_type=jnp.float32)


def _xw_matmul(x, W):
    n, ci = x.shape
    K, _, co = W.shape
    Wf = W.transpose(1, 0, 2).reshape(ci, K * co)
    bn = K * co
    bm = 128 if bn >= 16000 else 256
    bm = min(bm, n)
    npad = (-n) % bm
    if npad:
        x = jnp.pad(x, ((0, npad), (0, 0)))
    M = x.shape[0]
    out = pl.pallas_call(
        _xw_body,
        grid=(M // bm,),
        in_specs=[pl.BlockSpec((bm, ci), lambda i: (i, 0)),
                  pl.BlockSpec((ci, bn), lambda i: (0, 0))],
        out_specs=pl.BlockSpec((bm, bn), lambda i: (i, 0)),
        out_shape=jax.ShapeDtypeStruct((M, K * co), jnp.float32),
    )(x, Wf)
    return out[:n]


def _combine_body(x_ref, root_ref, b_ref, *refs, n, co_eff, S):
    t_refs, o_ref = refs[:S], refs[S]
    parts = []
    for h in range(S):
        t = t_refs[h][...]
        parts.append(t[0, :n, :co_eff] + t[1, :n, :co_eff])
    agg = jnp.concatenate(parts, axis=1) if S > 1 else parts[0]
    t0 = t_refs[0][...]
    deg = (t0[0, :n, co_eff] + t0[1, :n, co_eff])[:, None]
    z = agg / jnp.maximum(deg, 1.0)
    z = z + jnp.dot(x_ref[...], root_ref[...], preferred_element_type=jnp.float32)
    z = z + b_ref[...]
    o_ref[...] = jnp.where(z > 0, z, (jnp.exp(z) - 1.0))


def _combine(tables, x, root, bias, n, co_eff, S):
    n_pad, Wt = tables[0].shape[1], tables[0].shape[2]
    ci, co = root.shape
    return pl.pallas_call(
        functools.partial(_combine_body, n=n, co_eff=co_eff, S=S),
        in_specs=[pl.BlockSpec((n, ci), lambda: (0, 0)),
                  pl.BlockSpec((ci, co), lambda: (0, 0)),
                  pl.BlockSpec((1, co), lambda: (0, 0))]
        + [pl.BlockSpec((2, n_pad, Wt), lambda: (0, 0, 0))] * S,
        out_specs=pl.BlockSpec((n, co), lambda: (0, 0)),
        out_shape=jax.ShapeDtypeStruct((n, co), jnp.float32),
    )(x, root, bias[None, :], *tables)


def _head_body(z_ref, w1_ref, b1_ref, w2_ref, b2_ref, o_ref):
    z = z_ref[...].reshape(32, 1024)
    z = lax.dot_general(z, w1_ref[...], (((1,), (1,)), ((), ())),
                        preferred_element_type=jnp.float32) + b1_ref[...]
    z = jnp.where(z > 0, z, (jnp.exp(z) - 1.0))
    z = lax.dot_general(z, w2_ref[...], (((1,), (1,)), ((), ())),
                        preferred_element_type=jnp.float32) + b2_ref[...]
    m = jnp.max(z, axis=1, keepdims=True)
    s = z - m
    o_ref[...] = s - jnp.log(jnp.sum(jnp.exp(s), axis=1, keepdims=True))


def _head(hf, fc1_w, fc1_b, fc2_w, fc2_b):
    return pl.pallas_call(
        _head_body,
        in_specs=[pl.BlockSpec(hf.shape, lambda: (0, 0)),
                  pl.BlockSpec(fc1_w.shape, lambda: (0, 0)),
                  pl.BlockSpec((1, 512), lambda: (0, 0)),
                  pl.BlockSpec(fc2_w.shape, lambda: (0, 0)),
                  pl.BlockSpec((1, 10), lambda: (0, 0))],
        out_specs=pl.BlockSpec((32, 10), lambda: (0, 0)),
        out_shape=jax.ShapeDtypeStruct((32, 10), jnp.float32),
    )(hf, fc1_w, fc1_b[None, :], fc2_w, fc2_b[None, :])


def _graph_max_pool(x, pos, cluster, n_out):
    px = jax.ops.segment_max(x, cluster, num_segments=n_out)
    px = jnp.where(jnp.isfinite(px), px, 0.0)
    s = jax.ops.segment_sum(pos, cluster, num_segments=n_out)
    c = jax.ops.segment_sum(jnp.ones((pos.shape[0],), pos.dtype), cluster,
                            num_segments=n_out)
    return px, s / jnp.maximum(c, 1.0)[:, None]


def _n_pad(n):
    stripe = -(-n // 128) * 8
    return stripe * 16


def _spline_layer(h, src, dst, d3e, W, root, bias, n):
    K, ci, co = W.shape
    E = src.shape[0]
    S = 2 if co > 64 else 1
    co_eff = co // S
    metas = _edge_meta(d3e, src[None, :], dst[None, :], K, S)
    xw = _xw_matmul(h, W)
    xw2 = xw.reshape(n * K * S, co_eff)
    npad = _n_pad(n)
    sc_agg = _make_sc_edge_agg(npad, co_eff, E)
    tables = [sc_agg(xw2, metas[hh]) for hh in range(S)]
    return _combine(tables, h, root, bias, n, co_eff, S)


def kernel(x, pos, edge_index, cluster0, cluster1, cluster2, cluster3,
           cluster4, cluster5, W1, root1, b1, W2, root2, b2, W3, root3, b3,
           W4, root4, b4, W5, root5, b5, fc1_w, fc1_b, fc2_w, fc2_b):
    sizes = [5000, 2500, 1250, 640, 320]
    convs = [(W1, root1, b1), (W2, root2, b2), (W3, root3, b3),
             (W4, root4, b4), (W5, root5, b5)]
    clusters = [cluster0, cluster1, cluster2, cluster3, cluster4]
    h, p = x, pos
    ei32 = edge_index.astype(jnp.int32)
    src, dst = ei32[0], ei32[1]
    E = src.shape[0]
    prev_n = x.shape[0]
    for cl, n, (W, r, b) in zip(clusters, sizes, convs):
        cl = cl.astype(jnp.int32)
        h, p = _graph_max_pool(h, p, cl, n)
        geom = _make_sc_geometry(prev_n, n, E)
        src, dst, dflat = geom(cl, p.T, src, dst)
        h = _spline_layer(h, src, dst, dflat.reshape(3, E), W, r, b, n)
        prev_n = n
    hf, _ = _graph_max_pool(h, p, cluster5.astype(jnp.int32), 256)
    return _head(hf, fc1_w, fc1_b, fc2_w, fc2_b)


# speedup vs baseline: 15.0315x; 1.1054x over previous
import functools
import math

import jax
import jax.numpy as jnp
from jax import lax
from jax.experimental import pallas as pl
from jax.experimental.pallas import tpu as pltpu
from jax.experimental.pallas import tpu_sc as plsc

KS = 5
RADIX = (25, 5, 1)
OFFS = [(i, j, k) for i in (0, 1) for j in (0, 1) for k in (0, 1)]

LOG1P_SCALE = 30.0


def _edge_meta_body(d_ref, src_ref, dst_ref, *meta_refs, K, S, real_blocks,
                    inv_log1p_scale):
    d = d_ref[...]
    u = 0.5 + 0.5 * jnp.sign(d) * jnp.log1p(LOG1P_SCALE * jnp.abs(d)) * inv_log1p_scale
    u = jnp.clip(u, 0.0, 1.0)
    p = u * (KS - 1)
    bottom = jnp.clip(jnp.floor(p), 0.0, KS - 2)
    frac = p - bottom
    bot_i = bottom.astype(jnp.int32)
    b_rows = []
    k_rows = []
    for off in OFFS:
        b = jnp.ones_like(frac[0:1])
        k = jnp.zeros_like(bot_i[0:1])
        for dim in range(3):
            f = frac[dim:dim + 1]
            b = b * (f if off[dim] == 1 else (1.0 - f))
            k = k + (bot_i[dim:dim + 1] + off[dim]) * RADIX[dim]
        b_rows.append(b)
        k_rows.append(k)
    bas_i = lax.bitcast_convert_type(jnp.concatenate(b_rows, axis=0),
                                     jnp.int32)
    gi = jnp.concatenate(k_rows, axis=0) + src_ref[...] * K
    real = pl.program_id(0) < real_blocks
    for h in range(S):
        meta = jnp.concatenate([gi * S + h, bas_i, dst_ref[...]], axis=0)
        meta_refs[h][...] = jnp.where(real, meta, 0)


def _edge_meta(d3e, src, dst, K, S, E_pad):
    E = d3e.shape[1]
    blk = 1280
    real_blocks = E // blk
    grid = (E_pad // blk,)
    clamp = functools.partial(jnp.minimum, real_blocks - 1)
    outs = pl.pallas_call(
        functools.partial(_edge_meta_body, K=K, S=S,
                          real_blocks=real_blocks,
                          inv_log1p_scale=1.0 / math.log1p(LOG1P_SCALE)),
        grid=grid,
        in_specs=[pl.BlockSpec((3, blk), lambda i: (0, clamp(i))),
                  pl.BlockSpec((1, blk), lambda i: (0, clamp(i))),
                  pl.BlockSpec((1, blk), lambda i: (0, clamp(i)))],
        out_specs=[pl.BlockSpec((17, blk), lambda i: (0, i))] * S,
        out_shape=[jax.ShapeDtypeStruct((17, E_pad), jnp.int32)] * S,
    )(d3e, src, dst)
    return outs


def _make_sc_geometry(prev_n, n, E):
    C = 640
    NCHUNK = E // C
    mesh = plsc.VectorSubcoreMesh(core_axis_name="c", subcore_axis_name="s",
                                  num_cores=2, num_subcores=16)

    @functools.partial(
        pl.kernel,
        out_type=[jax.ShapeDtypeStruct((E,), jnp.int32),
                  jax.ShapeDtypeStruct((E,), jnp.int32),
                  jax.ShapeDtypeStruct((3 * E,), jnp.float32)],
        mesh=mesh,
        scratch_types=[
            pltpu.VMEM((prev_n,), jnp.int32),
            pltpu.VMEM((3, n), jnp.float32),
            pltpu.VMEM((C,), jnp.int32),
            pltpu.VMEM((C,), jnp.int32),
            pltpu.VMEM((C,), jnp.int32),
            pltpu.VMEM((C,), jnp.int32),
            pltpu.VMEM((3, C), jnp.float32),
        ],
        compiler_params=pltpu.CompilerParams(use_tc_tiling_on_sc=False,
                                             needs_layout_passes=False),
    )
    def k(cl_hbm, post_hbm, srcp_hbm, dstp_hbm, nsrc_hbm, ndst_hbm, d_hbm,
          cl_v, pos_v, sv, dv, nsv, ndv, dbuf):
        cid = lax.axis_index("c")
        sid = lax.axis_index("s")
        wid = sid * 2 + cid
        pltpu.sync_copy(cl_hbm, cl_v)
        pltpu.sync_copy(post_hbm, pos_v)
        nfull, rem = NCHUNK // 32, NCHUNK % 32
        ngroups = nfull + jnp.where(wid < rem, 1, 0)

        def chunk(g, carry):
            base = (wid + g * 32) * C
            pltpu.sync_copy(srcp_hbm.at[pl.ds(base, C)], sv)
            pltpu.sync_copy(dstp_hbm.at[pl.ds(base, C)], dv)
            for t in range(C // 16):
                sl = pl.ds(t * 16, 16)
                ns = plsc.load_gather(cl_v, [sv[sl]])
                nd = plsc.load_gather(cl_v, [dv[sl]])
                nsv[sl] = ns
                ndv[sl] = nd
                for dim in range(3):
                    dimv = jnp.full((16,), dim, jnp.int32)
                    ps = plsc.load_gather(pos_v, [dimv, ns])
                    pd = plsc.load_gather(pos_v, [dimv, nd])
                    dbuf[dim, sl] = pd - ps
            pltpu.sync_copy(nsv, nsrc_hbm.at[pl.ds(base, C)])
            pltpu.sync_copy(ndv, ndst_hbm.at[pl.ds(base, C)])
            for dim in range(3):
                pltpu.sync_copy(dbuf.at[dim], d_hbm.at[pl.ds(dim * E + base, C)])
            return carry
        lax.fori_loop(0, ngroups, chunk, 0)

    return k


def _make_sc_edge_agg(n_pad, co, E):
    W = co + 16
    G = co // 16
    C = 64
    NCHUNK = E // C
    NG = NCHUNK // 32
    stripe = n_pad // 16
    mesh = plsc.VectorSubcoreMesh(core_axis_name="c", subcore_axis_name="s",
                                  num_cores=2, num_subcores=16)

    @functools.partial(
        pl.kernel,
        out_type=jax.ShapeDtypeStruct((2, n_pad, W), jnp.float32),
        mesh=mesh,
        scratch_types=[
            pltpu.VMEM((17, C), jnp.int32),
            pltpu.VMEM((17, C), jnp.int32),
            pltpu.VMEM((8, C, co), jnp.float32),
            pltpu.VMEM((8, C, co), jnp.float32),
            pltpu.VMEM((C, W), jnp.float32),
            pltpu.VMEM((stripe, W), jnp.float32),
            pltpu.VMEM_SHARED((n_pad, W), jnp.float32),
            pltpu.SemaphoreType.DMA,
            pltpu.SemaphoreType.DMA,
        ],
        compiler_params=pltpu.CompilerParams(use_tc_tiling_on_sc=False,
                                             needs_layout_passes=False),
    )
    def k(xw_hbm, meta_hbm, out_hbm, m0, m1, r0, r1, msg_v, zero_v, agg_sh,
          sem0, sem1):
        cid = lax.axis_index("c")
        sid = lax.axis_index("s")
        wid = sid * 2 + cid

        zvec = jnp.zeros((16,), jnp.float32)

        def zrow(r, carry):
            for wg in range(W // 16):
                zero_v[r, pl.ds(wg * 16, 16)] = zvec
            return carry
        lax.fori_loop(0, stripe, zrow, 0)
        pltpu.sync_copy(zero_v, agg_sh.at[pl.ds(sid * stripe, stripe)])
        plsc.subcore_barrier()

        def fetch(g, mb, rb, sem):
            base = (wid + g * 32) * C
            pltpu.sync_copy(meta_hbm.at[:, pl.ds(base, C)], mb)
            for j in range(8):
                pltpu.async_copy(xw_hbm.at[mb.at[j]], rb.at[j], sem)

        def waitg(mb, rb, sem):
            for j in range(8):
                pltpu.make_async_copy(xw_hbm.at[mb.at[j]], rb.at[j],
                                      sem).wait()

        def compute(mb, rb):
            def group(t, c2):
                e0 = t * 16
                bvecs = [plsc.bitcast(mb[8 + j, pl.ds(e0, 16)], jnp.float32)
                         for j in range(8)]
                bsum = bvecs[0]
                for j in range(1, 8):
                    bsum = bsum + bvecs[j]
                for l in range(16):
                    e = e0 + l
                    for gg in range(G):
                        acc = bvecs[0][l] * rb[0, e, pl.ds(gg * 16, 16)]
                        for j in range(1, 8):
                            acc = acc + bvecs[j][l] * rb[j, e, pl.ds(gg * 16, 16)]
                        msg_v[e, pl.ds(gg * 16, 16)] = acc
                    msg_v[e, pl.ds(co, 16)] = zvec + bsum[l]
                return c2
            lax.fori_loop(0, C // 16, group, 0)
            pltpu.sync_copy(msg_v, agg_sh.at[mb.at[16]], add=True)

        fetch(0, m0, r0, sem0)

        def pair(p, carry):
            fetch(2 * p + 1, m1, r1, sem1)
            waitg(m0, r0, sem0)
            compute(m0, r0)

            @pl.when(p < NG // 2 - 1)
            def _prefetch():
                fetch(2 * p + 2, m0, r0, sem0)
            waitg(m1, r1, sem1)
            compute(m1, r1)
            return carry
        lax.fori_loop(0, NG // 2, pair, 0)

        plsc.subcore_barrier()
        @pl.when(sid == 0)
        def _dump():
            pltpu.sync_copy(agg_sh, out_hbm.at[cid])

    return k


def _xw_body(x_ref, w_ref, o_ref):
    o_ref[...] = jnp.dot(x_ref[...], w_ref[...],
                         preferred_element_type=jnp.float32)


def _xw_matmul(x, W):
    n, ci = x.shape
    K, _, co = W.shape
    Wf = W.transpose(1, 0, 2).reshape(ci, K * co)
    bn = K * co
    bm = 128 if bn >= 16000 else 256
    bm = min(bm, n)
    npad = (-n) % bm
    if npad:
        x = jnp.pad(x, ((0, npad), (0, 0)))
    M = x.shape[0]
    out = pl.pallas_call(
        _xw_body,
        grid=(M // bm,),
        in_specs=[pl.BlockSpec((bm, ci), lambda i: (i, 0)),
                  pl.BlockSpec((ci, bn), lambda i: (0, 0))],
        out_specs=pl.BlockSpec((bm, bn), lambda i: (i, 0)),
        out_shape=jax.ShapeDtypeStruct((M, K * co), jnp.float32),
    )(x, Wf)
    return out[:n]


def _combine_body(x_ref, root_ref, b_ref, *refs, n, co_eff, S):
    t_refs, o_ref = refs[:S], refs[S]
    parts = []
    for h in range(S):
        t = t_refs[h][...]
        parts.append(t[0, :n, :co_eff] + t[1, :n, :co_eff])
    agg = jnp.concatenate(parts, axis=1) if S > 1 else parts[0]
    t0 = t_refs[0][...]
    deg = (t0[0, :n, co_eff] + t0[1, :n, co_eff])[:, None]
    z = agg / jnp.maximum(deg, 1.0)
    z = z + jnp.dot(x_ref[...], root_ref[...], preferred_element_type=jnp.float32)
    z = z + b_ref[...]
    o_ref[...] = jnp.where(z > 0, z, (jnp.exp(z) - 1.0))


def _combine(tables, x, root, bias, n, co_eff, S):
    n_pad, Wt = tables[0].shape[1], tables[0].shape[2]
    ci, co = root.shape
    return pl.pallas_call(
        functools.partial(_combine_body, n=n, co_eff=co_eff, S=S),
        in_specs=[pl.BlockSpec((n, ci), lambda: (0, 0)),
                  pl.BlockSpec((ci, co), lambda: (0, 0)),
                  pl.BlockSpec((1, co), lambda: (0, 0))]
        + [pl.BlockSpec((2, n_pad, Wt), lambda: (0, 0, 0))] * S,
        out_specs=pl.BlockSpec((n, co), lambda: (0, 0)),
        out_shape=jax.ShapeDtypeStruct((n, co), jnp.float32),
    )(x, root, bias[None, :], *tables)


def _head_body(z_ref, w1_ref, b1_ref, w2_ref, b2_ref, o_ref):
    z = z_ref[...].reshape(32, 1024)
    z = lax.dot_general(z, w1_ref[...], (((1,), (1,)), ((), ())),
                        preferred_element_type=jnp.float32) + b1_ref[...]
    z = jnp.where(z > 0, z, (jnp.exp(z) - 1.0))
    z = lax.dot_general(z, w2_ref[...], (((1,), (1,)), ((), ())),
                        preferred_element_type=jnp.float32) + b2_ref[...]
    m = jnp.max(z, axis=1, keepdims=True)
    s = z - m
    o_ref[...] = s - jnp.log(jnp.sum(jnp.exp(s), axis=1, keepdims=True))


def _head(hf, fc1_w, fc1_b, fc2_w, fc2_b):
    return pl.pallas_call(
        _head_body,
        in_specs=[pl.BlockSpec(hf.shape, lambda: (0, 0)),
                  pl.BlockSpec(fc1_w.shape, lambda: (0, 0)),
                  pl.BlockSpec((1, 512), lambda: (0, 0)),
                  pl.BlockSpec(fc2_w.shape, lambda: (0, 0)),
                  pl.BlockSpec((1, 10), lambda: (0, 0))],
        out_specs=pl.BlockSpec((32, 10), lambda: (0, 0)),
        out_shape=jax.ShapeDtypeStruct((32, 10), jnp.float32),
    )(hf, fc1_w, fc1_b[None, :], fc2_w, fc2_b[None, :])


def _graph_max_pool(x, pos, cluster, n_out):
    px = jax.ops.segment_max(x, cluster, num_segments=n_out)
    px = jnp.where(jnp.isfinite(px), px, 0.0)
    s = jax.ops.segment_sum(pos, cluster, num_segments=n_out)
    c = jax.ops.segment_sum(jnp.ones((pos.shape[0],), pos.dtype), cluster,
                            num_segments=n_out)
    return px, s / jnp.maximum(c, 1.0)[:, None]


def _n_pad(n):
    stripe = -(-n // 128) * 8
    return stripe * 16


def _spline_layer(h, src, dst, d3e, W, root, bias, n):
    K, ci, co = W.shape
    E = src.shape[0]
    S = 2 if co > 64 else 1
    co_eff = co // S
    E_pad = -(-E // 2048) * 2048
    metas = _edge_meta(d3e, src[None, :], dst[None, :], K, S, E_pad)
    xw = _xw_matmul(h, W)
    xw2 = xw.reshape(n * K * S, co_eff)
    npad = _n_pad(n)
    sc_agg = _make_sc_edge_agg(npad, co_eff, E_pad)
    tables = [sc_agg(xw2, metas[hh]) for hh in range(S)]
    return _combine(tables, h, root, bias, n, co_eff, S)


def kernel(x, pos, edge_index, cluster0, cluster1, cluster2, cluster3,
           cluster4, cluster5, W1, root1, b1, W2, root2, b2, W3, root3, b3,
           W4, root4, b4, W5, root5, b5, fc1_w, fc1_b, fc2_w, fc2_b):
    sizes = [5000, 2500, 1250, 640, 320]
    convs = [(W1, root1, b1), (W2, root2, b2), (W3, root3, b3),
             (W4, root4, b4), (W5, root5, b5)]
    clusters = [cluster0, cluster1, cluster2, cluster3, cluster4]
    h, p = x, pos
    ei32 = edge_index.astype(jnp.int32)
    src, dst = ei32[0], ei32[1]
    E = src.shape[0]
    prev_n = x.shape[0]
    for cl, n, (W, r, b) in zip(clusters, sizes, convs):
        cl = cl.astype(jnp.int32)
        h, p = _graph_max_pool(h, p, cl, n)
        geom = _make_sc_geometry(prev_n, n, E)
        src, dst, dflat = geom(cl, p.T, src, dst)
        h = _spline_layer(h, src, dst, dflat.reshape(3, E), W, r, b, n)
        prev_n = n
    hf, _ = _graph_max_pool(h, p, cluster5.astype(jnp.int32), 256)
    return _head(hf, fc1_w, fc1_b, fc2_w, fc2_b)


# speedup vs baseline: 15.6668x; 1.0423x over previous
import functools
import math

import jax
import jax.numpy as jnp
from jax import lax
from jax.experimental import pallas as pl
from jax.experimental.pallas import tpu as pltpu
from jax.experimental.pallas import tpu_sc as plsc

KS = 5
RADIX = (25, 5, 1)
OFFS = [(i, j, k) for i in (0, 1) for j in (0, 1) for k in (0, 1)]

LOG1P_SCALE = 30.0


def _edge_meta_body(d_ref, src_ref, dst_ref, *meta_refs, K, S, real_blocks,
                    inv_log1p_scale):
    d = d_ref[...]
    u = 0.5 + 0.5 * jnp.sign(d) * jnp.log1p(LOG1P_SCALE * jnp.abs(d)) * inv_log1p_scale
    u = jnp.clip(u, 0.0, 1.0)
    p = u * (KS - 1)
    bottom = jnp.clip(jnp.floor(p), 0.0, KS - 2)
    frac = p - bottom
    bot_i = bottom.astype(jnp.int32)
    b_rows = []
    k_rows = []
    for off in OFFS:
        b = jnp.ones_like(frac[0:1])
        k = jnp.zeros_like(bot_i[0:1])
        for dim in range(3):
            f = frac[dim:dim + 1]
            b = b * (f if off[dim] == 1 else (1.0 - f))
            k = k + (bot_i[dim:dim + 1] + off[dim]) * RADIX[dim]
        b_rows.append(b)
        k_rows.append(k)
    bas_i = lax.bitcast_convert_type(jnp.concatenate(b_rows, axis=0),
                                     jnp.int32)
    gi = jnp.concatenate(k_rows, axis=0) + src_ref[...] * K
    real = pl.program_id(0) < real_blocks
    for h in range(S):
        meta = jnp.concatenate([gi * S + h, bas_i, dst_ref[...]], axis=0)
        meta_refs[h][...] = jnp.where(real, meta, 0)


def _edge_meta(d3e, src, dst, K, S, E_pad):
    E = d3e.shape[1]
    blk = 1280
    real_blocks = E // blk
    grid = (E_pad // blk,)
    clamp = functools.partial(jnp.minimum, real_blocks - 1)
    outs = pl.pallas_call(
        functools.partial(_edge_meta_body, K=K, S=S,
                          real_blocks=real_blocks,
                          inv_log1p_scale=1.0 / math.log1p(LOG1P_SCALE)),
        grid=grid,
        in_specs=[pl.BlockSpec((3, blk), lambda i: (0, clamp(i))),
                  pl.BlockSpec((1, blk), lambda i: (0, clamp(i))),
                  pl.BlockSpec((1, blk), lambda i: (0, clamp(i)))],
        out_specs=[pl.BlockSpec((17, blk), lambda i: (0, i))] * S,
        out_shape=[jax.ShapeDtypeStruct((17, E_pad), jnp.int32)] * S,
    )(d3e, src, dst)
    return outs


def _make_sc_geometry(prev_n, n, E):
    C = 640
    NCHUNK = E // C
    mesh = plsc.VectorSubcoreMesh(core_axis_name="c", subcore_axis_name="s",
                                  num_cores=2, num_subcores=16)

    @functools.partial(
        pl.kernel,
        out_type=[jax.ShapeDtypeStruct((E,), jnp.int32),
                  jax.ShapeDtypeStruct((E,), jnp.int32),
                  jax.ShapeDtypeStruct((3 * E,), jnp.float32)],
        mesh=mesh,
        scratch_types=[
            pltpu.VMEM((prev_n,), jnp.int32),
            pltpu.VMEM((3, n), jnp.float32),
            pltpu.VMEM((C,), jnp.int32),
            pltpu.VMEM((C,), jnp.int32),
            pltpu.VMEM((C,), jnp.int32),
            pltpu.VMEM((C,), jnp.int32),
            pltpu.VMEM((3, C), jnp.float32),
        ],
        compiler_params=pltpu.CompilerParams(use_tc_tiling_on_sc=False,
                                             needs_layout_passes=False),
    )
    def k(cl_hbm, post_hbm, srcp_hbm, dstp_hbm, nsrc_hbm, ndst_hbm, d_hbm,
          cl_v, pos_v, sv, dv, nsv, ndv, dbuf):
        cid = lax.axis_index("c")
        sid = lax.axis_index("s")
        wid = sid * 2 + cid
        pltpu.sync_copy(cl_hbm, cl_v)
        pltpu.sync_copy(post_hbm, pos_v)
        nfull, rem = NCHUNK // 32, NCHUNK % 32
        ngroups = nfull + jnp.where(wid < rem, 1, 0)

        def chunk(g, carry):
            base = (wid + g * 32) * C
            pltpu.sync_copy(srcp_hbm.at[pl.ds(base, C)], sv)
            pltpu.sync_copy(dstp_hbm.at[pl.ds(base, C)], dv)
            for t in range(C // 16):
                sl = pl.ds(t * 16, 16)
                ns = plsc.load_gather(cl_v, [sv[sl]])
                nd = plsc.load_gather(cl_v, [dv[sl]])
                nsv[sl] = ns
                ndv[sl] = nd
                for dim in range(3):
                    dimv = jnp.full((16,), dim, jnp.int32)
                    ps = plsc.load_gather(pos_v, [dimv, ns])
                    pd = plsc.load_gather(pos_v, [dimv, nd])
                    dbuf[dim, sl] = pd - ps
            pltpu.sync_copy(nsv, nsrc_hbm.at[pl.ds(base, C)])
            pltpu.sync_copy(ndv, ndst_hbm.at[pl.ds(base, C)])
            for dim in range(3):
                pltpu.sync_copy(dbuf.at[dim], d_hbm.at[pl.ds(dim * E + base, C)])
            return carry
        lax.fori_loop(0, ngroups, chunk, 0)

    return k


def _make_sc_edge_agg(n_pad, co, E):
    W = co + 16
    G = co // 16
    C = 64
    NCHUNK = E // C
    NG = NCHUNK // 32
    stripe = n_pad // 16
    mesh = plsc.VectorSubcoreMesh(core_axis_name="c", subcore_axis_name="s",
                                  num_cores=2, num_subcores=16)

    @functools.partial(
        pl.kernel,
        out_type=jax.ShapeDtypeStruct((2, n_pad, W), jnp.float32),
        mesh=mesh,
        scratch_types=[
            pltpu.VMEM((17, C), jnp.int32),
            pltpu.VMEM((17, C), jnp.int32),
            pltpu.VMEM((8, C, co), jnp.float32),
            pltpu.VMEM((8, C, co), jnp.float32),
            pltpu.VMEM((C, W), jnp.float32),
            pltpu.VMEM((stripe, W), jnp.float32),
            pltpu.VMEM_SHARED((n_pad, W), jnp.float32),
            pltpu.SemaphoreType.DMA,
            pltpu.SemaphoreType.DMA,
        ],
        compiler_params=pltpu.CompilerParams(use_tc_tiling_on_sc=False,
                                             needs_layout_passes=False),
    )
    def k(xw_hbm, meta_hbm, out_hbm, m0, m1, r0, r1, msg_v, zero_v, agg_sh,
          sem0, sem1):
        cid = lax.axis_index("c")
        sid = lax.axis_index("s")
        wid = sid * 2 + cid

        zvec = jnp.zeros((16,), jnp.float32)

        def zrow(r, carry):
            for wg in range(W // 16):
                zero_v[r, pl.ds(wg * 16, 16)] = zvec
            return carry
        lax.fori_loop(0, stripe, zrow, 0)
        pltpu.sync_copy(zero_v, agg_sh.at[pl.ds(sid * stripe, stripe)])
        plsc.subcore_barrier()

        def fetch(g, mb, rb, sem):
            base = (wid + g * 32) * C
            pltpu.sync_copy(meta_hbm.at[:, pl.ds(base, C)], mb)
            for j in range(8):
                pltpu.async_copy(xw_hbm.at[mb.at[j]], rb.at[j], sem)

        def waitg(mb, rb, sem):
            for j in range(8):
                pltpu.make_async_copy(xw_hbm.at[mb.at[j]], rb.at[j],
                                      sem).wait()

        def compute(mb, rb):
            def group(t, c2):
                e0 = t * 16
                bvecs = [plsc.bitcast(mb[8 + j, pl.ds(e0, 16)], jnp.float32)
                         for j in range(8)]
                bsum = bvecs[0]
                for j in range(1, 8):
                    bsum = bsum + bvecs[j]
                for l in range(16):
                    e = e0 + l
                    bcast = [zvec + bvecs[j][l] for j in range(8)]
                    for gg in range(G):
                        acc = bcast[0] * rb[0, e, pl.ds(gg * 16, 16)]
                        for j in range(1, 8):
                            acc = acc + bcast[j] * rb[j, e, pl.ds(gg * 16, 16)]
                        msg_v[e, pl.ds(gg * 16, 16)] = acc
                    msg_v[e, pl.ds(co, 16)] = zvec + bsum[l]
                return c2
            lax.fori_loop(0, C // 16, group, 0)
            pltpu.sync_copy(msg_v, agg_sh.at[mb.at[16]], add=True)

        fetch(0, m0, r0, sem0)

        def pair(p, carry):
            fetch(2 * p + 1, m1, r1, sem1)
            waitg(m0, r0, sem0)
            compute(m0, r0)

            @pl.when(p < NG // 2 - 1)
            def _prefetch():
                fetch(2 * p + 2, m0, r0, sem0)
            waitg(m1, r1, sem1)
            compute(m1, r1)
            return carry
        lax.fori_loop(0, NG // 2, pair, 0)

        plsc.subcore_barrier()
        @pl.when(sid == 0)
        def _dump():
            pltpu.sync_copy(agg_sh, out_hbm.at[cid])

    return k


def _xw_body(x_ref, w_ref, o_ref):
    o_ref[...] = jnp.dot(x_ref[...], w_ref[...],
                         preferred_element_type=jnp.float32)


def _xw_matmul(x, W):
    n, ci = x.shape
    K, _, co = W.shape
    Wf = W.transpose(1, 0, 2).reshape(ci, K * co)
    bn = K * co
    bm = 128 if bn >= 16000 else 256
    bm = min(bm, n)
    npad = (-n) % bm
    if npad:
        x = jnp.pad(x, ((0, npad), (0, 0)))
    M = x.shape[0]
    out = pl.pallas_call(
        _xw_body,
        grid=(M // bm,),
        in_specs=[pl.BlockSpec((bm, ci), lambda i: (i, 0)),
                  pl.BlockSpec((ci, bn), lambda i: (0, 0))],
        out_specs=pl.BlockSpec((bm, bn), lambda i: (i, 0)),
        out_shape=jax.ShapeDtypeStruct((M, K * co), jnp.float32),
    )(x, Wf)
    return out


def _combine_body(x_ref, root_ref, b_ref, *refs, n, co_eff, S):
    t_refs, o_ref = refs[:S], refs[S]
    parts = []
    for h in range(S):
        t = t_refs[h][...]
        parts.append(t[0, :n, :co_eff] + t[1, :n, :co_eff])
    agg = jnp.concatenate(parts, axis=1) if S > 1 else parts[0]
    t0 = t_refs[0][...]
    deg = (t0[0, :n, co_eff] + t0[1, :n, co_eff])[:, None]
    z = agg / jnp.maximum(deg, 1.0)
    z = z + jnp.dot(x_ref[...], root_ref[...], preferred_element_type=jnp.float32)
    z = z + b_ref[...]
    o_ref[...] = jnp.where(z > 0, z, (jnp.exp(z) - 1.0))


def _combine(tables, x, root, bias, n, co_eff, S):
    n_pad, Wt = tables[0].shape[1], tables[0].shape[2]
    ci, co = root.shape
    return pl.pallas_call(
        functools.partial(_combine_body, n=n, co_eff=co_eff, S=S),
        in_specs=[pl.BlockSpec((n, ci), lambda: (0, 0)),
                  pl.BlockSpec((ci, co), lambda: (0, 0)),
                  pl.BlockSpec((1, co), lambda: (0, 0))]
        + [pl.BlockSpec((2, n_pad, Wt), lambda: (0, 0, 0))] * S,
        out_specs=pl.BlockSpec((n, co), lambda: (0, 0)),
        out_shape=jax.ShapeDtypeStruct((n, co), jnp.float32),
    )(x, root, bias[None, :], *tables)


def _head_body(z_ref, w1_ref, b1_ref, w2_ref, b2_ref, o_ref):
    z = z_ref[...].reshape(32, 1024)
    z = lax.dot_general(z, w1_ref[...], (((1,), (1,)), ((), ())),
                        preferred_element_type=jnp.float32) + b1_ref[...]
    z = jnp.where(z > 0, z, (jnp.exp(z) - 1.0))
    z = lax.dot_general(z, w2_ref[...], (((1,), (1,)), ((), ())),
                        preferred_element_type=jnp.float32) + b2_ref[...]
    m = jnp.max(z, axis=1, keepdims=True)
    s = z - m
    o_ref[...] = s - jnp.log(jnp.sum(jnp.exp(s), axis=1, keepdims=True))


def _head(hf, fc1_w, fc1_b, fc2_w, fc2_b):
    return pl.pallas_call(
        _head_body,
        in_specs=[pl.BlockSpec(hf.shape, lambda: (0, 0)),
                  pl.BlockSpec(fc1_w.shape, lambda: (0, 0)),
                  pl.BlockSpec((1, 512), lambda: (0, 0)),
                  pl.BlockSpec(fc2_w.shape, lambda: (0, 0)),
                  pl.BlockSpec((1, 10), lambda: (0, 0))],
        out_specs=pl.BlockSpec((32, 10), lambda: (0, 0)),
        out_shape=jax.ShapeDtypeStruct((32, 10), jnp.float32),
    )(hf, fc1_w, fc1_b[None, :], fc2_w, fc2_b[None, :])


def _graph_max_pool(x, pos, cluster, n_out):
    px = jax.ops.segment_max(x, cluster, num_segments=n_out)
    px = jnp.where(jnp.isfinite(px), px, 0.0)
    s = jax.ops.segment_sum(pos, cluster, num_segments=n_out)
    c = jax.ops.segment_sum(jnp.ones((pos.shape[0],), pos.dtype), cluster,
                            num_segments=n_out)
    return px, s / jnp.maximum(c, 1.0)[:, None]


def _n_pad(n):
    stripe = -(-n // 128) * 8
    return stripe * 16


def _spline_layer(h, src, dst, d3e, W, root, bias, n):
    K, ci, co = W.shape
    E = src.shape[0]
    S = 2 if co > 64 else 1
    co_eff = co // S
    E_pad = -(-E // 2048) * 2048
    metas = _edge_meta(d3e, src[None, :], dst[None, :], K, S, E_pad)
    xw = _xw_matmul(h, W)
    xw2 = xw.reshape(xw.shape[0] * K * S, co_eff)
    npad = _n_pad(n)
    sc_agg = _make_sc_edge_agg(npad, co_eff, E_pad)
    tables = [sc_agg(xw2, metas[hh]) for hh in range(S)]
    return _combine(tables, h, root, bias, n, co_eff, S)


def kernel(x, pos, edge_index, cluster0, cluster1, cluster2, cluster3,
           cluster4, cluster5, W1, root1, b1, W2, root2, b2, W3, root3, b3,
           W4, root4, b4, W5, root5, b5, fc1_w, fc1_b, fc2_w, fc2_b):
    sizes = [5000, 2500, 1250, 640, 320]
    convs = [(W1, root1, b1), (W2, root2, b2), (W3, root3, b3),
             (W4, root4, b4), (W5, root5, b5)]
    clusters = [cluster0, cluster1, cluster2, cluster3, cluster4]
    h, p = x, pos
    ei32 = edge_index.astype(jnp.int32)
    src, dst = ei32[0], ei32[1]
    E = src.shape[0]
    prev_n = x.shape[0]
    for cl, n, (W, r, b) in zip(clusters, sizes, convs):
        cl = cl.astype(jnp.int32)
        h, p = _graph_max_pool(h, p, cl, n)
        geom = _make_sc_geometry(prev_n, n, E)
        src, dst, dflat = geom(cl, p.T, src, dst)
        h = _spline_layer(h, src, dst, dflat.reshape(3, E), W, r, b, n)
        prev_n = n
    hf, _ = _graph_max_pool(h, p, cluster5.astype(jnp.int32), 256)
    return _head(hf, fc1_w, fc1_b, fc2_w, fc2_b)
